# baseline (speedup 1.0000x reference)
import jax
import jax.numpy as jnp
from jax.experimental import pallas as pl

N = 32768
H1, H2, NS = 64, 32, 16
NB = 4
CB = N // NB

_C00 = (((0,), (0,)), ((), ()))


def _mlp_block(rx_ref, w1_ref, b1_ref, w2_ref, b2_ref, w3_ref, b3_ref, out_ref):
    f32 = jnp.float32
    a1 = jnp.concatenate([w1_ref[...], b1_ref[...]], axis=0)
    e1 = jnp.concatenate([jnp.zeros((1, 1), f32), jnp.ones((1, 1), f32)], axis=0)
    a1p = jnp.concatenate([a1, e1], axis=1)
    a2 = jnp.concatenate([w2_ref[...], b2_ref[...]], axis=0)
    e2 = jnp.concatenate([jnp.zeros((H1, 1), f32), jnp.ones((1, 1), f32)], axis=0)
    a2p = jnp.concatenate([a2, e2], axis=1)
    a3 = jnp.concatenate([w3_ref[...], b3_ref[...]], axis=0)

    xp = jnp.concatenate([rx_ref[0], jnp.ones((1, CB), f32)], axis=0)
    h1 = jax.lax.dot_general(a1p, xp, _C00,
                             preferred_element_type=f32)
    h1 = jnp.maximum(h1, 0.0)
    h2 = jax.lax.dot_general(a2p, h1, _C00,
                             preferred_element_type=f32)
    h2 = jnp.maximum(h2, 0.0)
    out_ref[...] = jax.lax.dot_general(a3, h2, _C00,
                                       preferred_element_type=f32)


def kernel(rx, W1, b1, W2, b2, W3, b3):
    rxr = rx.reshape(NB, 1, CB)
    res = pl.pallas_call(
        _mlp_block,
        grid=(NB,),
        in_specs=[
            pl.BlockSpec((1, 1, CB), lambda i: (i, 0, 0)),
            pl.BlockSpec((1, H1), lambda i: (0, 0)),
            pl.BlockSpec((1, H1), lambda i: (0, 0)),
            pl.BlockSpec((H1, H2), lambda i: (0, 0)),
            pl.BlockSpec((1, H2), lambda i: (0, 0)),
            pl.BlockSpec((H2, NS), lambda i: (0, 0)),
            pl.BlockSpec((1, NS), lambda i: (0, 0)),
        ],
        out_specs=pl.BlockSpec((NS, CB), lambda i: (0, i)),
        out_shape=jax.ShapeDtypeStruct((NS, N), jnp.float32),
    )(rxr, W1, b1.reshape(1, H1), W2, b2.reshape(1, H2), W3, b3.reshape(1, NS))
    return res.T


# speedup vs baseline: 1.0671x; 1.0671x over previous
import jax
import jax.numpy as jnp
from jax.experimental import pallas as pl

N = 32768
H1, H2, NS = 64, 32, 16
NB = 4
CB = N // NB

_C00 = (((0,), (0,)), ((), ()))


def _mlp_block(rx_ref, w1_ref, b1_ref, w2_ref, b2_ref, w3_ref, b3_ref, out_ref):
    f32 = jnp.float32
    w1c = jnp.concatenate([w1_ref[...].T, jnp.zeros((1, 1), f32)], axis=0)
    b1c = jnp.concatenate([b1_ref[...].T, jnp.ones((1, 1), f32)], axis=0)
    a2 = jnp.concatenate([w2_ref[...], b2_ref[...]], axis=0)
    e2 = jnp.concatenate([jnp.zeros((H1, 1), f32), jnp.ones((1, 1), f32)], axis=0)
    a2p = jnp.concatenate([a2, e2], axis=1)
    a3 = jnp.concatenate([w3_ref[...], b3_ref[...]], axis=0)

    x = rx_ref[0]
    h1 = jnp.maximum(w1c * x + b1c, 0.0)
    h2 = jax.lax.dot_general(a2p, h1, _C00,
                             preferred_element_type=f32)
    h2 = jnp.maximum(h2, 0.0)
    out_ref[...] = jax.lax.dot_general(a3, h2, _C00,
                                       preferred_element_type=f32)


def kernel(rx, W1, b1, W2, b2, W3, b3):
    rxr = rx.reshape(NB, 1, CB)
    res = pl.pallas_call(
        _mlp_block,
        grid=(NB,),
        in_specs=[
            pl.BlockSpec((1, 1, CB), lambda i: (i, 0, 0)),
            pl.BlockSpec((1, H1), lambda i: (0, 0)),
            pl.BlockSpec((1, H1), lambda i: (0, 0)),
            pl.BlockSpec((H1, H2), lambda i: (0, 0)),
            pl.BlockSpec((1, H2), lambda i: (0, 0)),
            pl.BlockSpec((H2, NS), lambda i: (0, 0)),
            pl.BlockSpec((1, NS), lambda i: (0, 0)),
        ],
        out_specs=pl.BlockSpec((NS, CB), lambda i: (0, i)),
        out_shape=jax.ShapeDtypeStruct((NS, N), jnp.float32),
    )(rxr, W1, b1.reshape(1, H1), W2, b2.reshape(1, H2), W3, b3.reshape(1, NS))
    return res.T


# speedup vs baseline: 1.1660x; 1.0928x over previous
import jax
import jax.numpy as jnp
from jax.experimental import pallas as pl

N = 32768
H1, H2, NS = 64, 32, 16
NB = 2
CB = N // NB

_C00 = (((0,), (0,)), ((), ()))


def _mlp_block(rx_ref, w1_ref, b1_ref, w2_ref, b2_ref, w3_ref, b3_ref, out_ref):
    f32 = jnp.float32
    w1c = jnp.concatenate([w1_ref[...].T, jnp.zeros((1, 1), f32)], axis=0)
    b1c = jnp.concatenate([b1_ref[...].T, jnp.ones((1, 1), f32)], axis=0)
    a2 = jnp.concatenate([w2_ref[...], b2_ref[...]], axis=0)
    e2 = jnp.concatenate([jnp.zeros((H1, 1), f32), jnp.ones((1, 1), f32)], axis=0)
    a2p = jnp.concatenate([a2, e2], axis=1)
    a3 = jnp.concatenate([w3_ref[...], b3_ref[...]], axis=0)

    x = rx_ref[0]
    h1 = jnp.maximum(w1c * x + b1c, 0.0)
    h2 = jax.lax.dot_general(a2p, h1, _C00,
                             preferred_element_type=f32)
    h2 = jnp.maximum(h2, 0.0)
    out_ref[...] = jax.lax.dot_general(a3, h2, _C00,
                                       preferred_element_type=f32)


def kernel(rx, W1, b1, W2, b2, W3, b3):
    rxr = rx.reshape(NB, 1, CB)
    res = pl.pallas_call(
        _mlp_block,
        grid=(NB,),
        in_specs=[
            pl.BlockSpec((1, 1, CB), lambda i: (i, 0, 0)),
            pl.BlockSpec((1, H1), lambda i: (0, 0)),
            pl.BlockSpec((1, H1), lambda i: (0, 0)),
            pl.BlockSpec((H1, H2), lambda i: (0, 0)),
            pl.BlockSpec((1, H2), lambda i: (0, 0)),
            pl.BlockSpec((H2, NS), lambda i: (0, 0)),
            pl.BlockSpec((1, NS), lambda i: (0, 0)),
        ],
        out_specs=pl.BlockSpec((NS, CB), lambda i: (0, i)),
        out_shape=jax.ShapeDtypeStruct((NS, N), jnp.float32),
    )(rxr, W1, b1.reshape(1, H1), W2, b2.reshape(1, H2), W3, b3.reshape(1, NS))
    return res.T
